# baseline (speedup 1.0000x reference)
import functools

import numpy as np
import jax
import jax.numpy as jnp
from jax import lax
from jax.experimental import pallas as pl
from jax.experimental.pallas import tpu as pltpu
from jax.experimental.pallas import tpu_sc as plsc

VOCAB_SIZE = 1000000
D_DIM = 64
BATCH_N = 4096
SEQ_L = 200


def _pe_table() -> np.ndarray:
    pos = np.arange(SEQ_L)[:, np.newaxis].astype(np.float64)
    i = np.arange(D_DIM)[np.newaxis, :].astype(np.float64)
    angle_rates = 1.0 / np.power(10000, 2 * (i // 2) / np.float32(D_DIM))
    angle_rads = pos * angle_rates
    angle_rads[:, 0::2] = np.sin(angle_rads[:, 0::2])
    angle_rads[:, 1::2] = np.cos(angle_rads[:, 1::2])
    return np.asarray(angle_rads, dtype=np.float32)


_PE_CONST = _pe_table()

_INFO = plsc.get_sparse_core_info()
_NC, _NS = _INFO.num_cores, _INFO.num_subcores
NW = _NC * _NS

SEQ_PER_W = BATCH_N // NW
SEQ_PER_CHUNK = 2
CHUNK = SEQ_PER_CHUNK * SEQ_L
NCHUNK = SEQ_PER_W // SEQ_PER_CHUNK
_SUBS = [(0, 128), (128, 72)]
LANES = 16
VECS_PER_ROW = D_DIM // LANES


def _sc_embed(table2, idx2d, pe):
    mesh = plsc.VectorSubcoreMesh(core_axis_name="c", subcore_axis_name="s")

    @functools.partial(
        pl.kernel,
        mesh=mesh,
        out_type=jax.ShapeDtypeStruct((BATCH_N, SEQ_L, D_DIM), jnp.float32),
        scratch_types=[
            pltpu.VMEM((SEQ_PER_CHUNK, SEQ_L), jnp.int32),
            pltpu.VMEM((CHUNK, 2 * D_DIM), jnp.float32),
            pltpu.VMEM((SEQ_PER_CHUNK, SEQ_L, D_DIM), jnp.float32),
            pltpu.VMEM((SEQ_L, D_DIM), jnp.float32),
            pltpu.SemaphoreType.DMA,
        ],
        compiler_params=pltpu.CompilerParams(use_tc_tiling_on_sc=True),
    )
    def body(table_hbm, idx_hbm, pe_hbm, out_hbm,
             idx_v, rows_v, out_v, pe_v, sem):
        wid = lax.axis_index("s") * _NC + lax.axis_index("c")
        seq_base = wid * SEQ_PER_W
        pltpu.sync_copy(pe_hbm, pe_v)

        def chunk_body(g, carry):
            seq0 = pl.multiple_of(seq_base + g * SEQ_PER_CHUNK, SEQ_PER_CHUNK)
            pltpu.sync_copy(idx_hbm.at[pl.ds(seq0, SEQ_PER_CHUNK)], idx_v)
            copies = []
            for s in range(SEQ_PER_CHUNK):
                for off, ln in _SUBS:
                    copies.append(
                        pltpu.async_copy(
                            table_hbm.at[idx_v.at[s].at[pl.ds(off, ln)]],
                            rows_v.at[pl.ds(s * SEQ_L + off, ln)],
                            sem,
                        )
                    )
            for cp in copies:
                cp.wait()

            def add_body(r, c2):
                for s in range(SEQ_PER_CHUNK):
                    row = s * SEQ_L + r
                    for j in range(VECS_PER_ROW):
                        sl = pl.ds(j * LANES, LANES)
                        out_v[s, r, sl] = rows_v[row, sl] + pe_v[r, sl]
                return c2

            lax.fori_loop(0, SEQ_L, add_body, 0)
            pltpu.sync_copy(out_v, out_hbm.at[pl.ds(seq0, SEQ_PER_CHUNK)])
            return carry

        lax.fori_loop(0, NCHUNK, chunk_body, 0)

    return body(table2, idx2d, pe)


def kernel(inputs, table):
    table2 = jnp.concatenate([table, table], axis=1)
    pe = jnp.asarray(_PE_CONST)
    return _sc_embed(table2, inputs, pe)
